# baseline (speedup 1.0000x reference)
import functools

import jax
import jax.numpy as jnp
from jax import lax
from jax.experimental import pallas as pl
from jax.experimental.pallas import tpu as pltpu
from jax.experimental.pallas import tpu_sc as plsc

N = 10000
E = 320000
D_IN = 128
H = 32
D_OUT = 128

NC = 2
NS = 16
NW = NC * NS
MC = 128
NCH = 80
EPW = NCH * MC
E_PAD = NW * EPW
N_PAD = 10240
RPT = N_PAD // NS

BLK = 1280
GRID = N_PAD // BLK

_mesh = plsc.VectorSubcoreMesh(core_axis_name="c", subcore_axis_name="s")
_sc_params = pltpu.CompilerParams(use_tc_tiling_on_sc=False)


def _zero16():
    return jnp.zeros((16,), jnp.float32)


def _sc_edge1_body(tab, srcs, dsts, vs, aggp, cntp,
                   src_vm, dst_vm, v_vm, rows0, rows1, msg0, msg1, ones, zer16,
                   sem0, sem1, smsg0, smsg1, sones, acc_sh, cnt_sh):
    c = lax.axis_index("c")
    s = lax.axis_index("s")
    wid = s * NC + c
    rows_bufs = (rows0, rows1)
    msg_bufs = (msg0, msg1)
    sems = (sem0, sem1)
    smsgs = (smsg0, smsg1)
    msg = msg0

    one16 = jnp.ones((16,), jnp.float32)

    def init_row(i, _):
        msg[i, pl.ds(0, 16)] = _zero16()
        msg[i, pl.ds(16, 16)] = _zero16()
        ones[i, pl.ds(0, 16)] = one16
        zer16[i, pl.ds(0, 16)] = _zero16()
        return 0
    lax.fori_loop(0, MC, init_row, 0)

    for r in range(RPT // MC):
        base = s * RPT + r * MC
        pltpu.sync_copy(msg, acc_sh.at[pl.ds(base, MC)])
        pltpu.sync_copy(zer16, cnt_sh.at[pl.ds(base, MC)])
    plsc.subcore_barrier()

    pltpu.sync_copy(srcs.at[wid], src_vm)
    pltpu.sync_copy(dsts.at[wid], dst_vm)
    pltpu.sync_copy(vs.at[wid], v_vm)

    pltpu.async_copy(tab.at[src_vm.at[0]], rows0, sem0)

    def pair(t, _):
        for b in range(2):
            j = t * 2 + b
            rows = rows_bufs[b]
            msg = msg_bufs[b]
            pltpu.make_async_copy(tab.at[src_vm.at[j]], rows, sems[b]).wait()

            @pl.when(j + 1 < NCH)
            def _prefetch():
                pltpu.async_copy(tab.at[src_vm.at[j + 1]],
                                 rows_bufs[1 - b], sems[1 - b])


        return 0
    lax.fori_loop(0, NCH // 2, pair, 0)

    plsc.subcore_barrier()
    pltpu.sync_copy(acc_sh.at[pl.ds(s * RPT, RPT)], aggp.at[c, pl.ds(s * RPT, RPT)])
    pltpu.sync_copy(cnt_sh.at[pl.ds(s * RPT, RPT)], cntp.at[c, pl.ds(s * RPT, RPT)])


_sc_edge1 = pl.kernel(
    _sc_edge1_body,
    out_type=(jax.ShapeDtypeStruct((NC, N_PAD, 32), jnp.float32),
              jax.ShapeDtypeStruct((NC, N_PAD, 16), jnp.float32)),
    mesh=_mesh,
    scratch_types=[
        pltpu.VMEM((NCH, MC), jnp.int32),
        pltpu.VMEM((NCH, MC), jnp.int32),
        pltpu.VMEM((NCH, MC), jnp.float32),
        pltpu.VMEM((MC, 64), jnp.float32),
        pltpu.VMEM((MC, 64), jnp.float32),
        pltpu.VMEM((MC, 32), jnp.float32),
        pltpu.VMEM((MC, 32), jnp.float32),
        pltpu.VMEM((MC, 16), jnp.float32),
        pltpu.VMEM((MC, 16), jnp.float32),
        pltpu.SemaphoreType.DMA,
        pltpu.SemaphoreType.DMA,
        pltpu.SemaphoreType.DMA,
        pltpu.SemaphoreType.DMA,
        pltpu.SemaphoreType.DMA,
        pltpu.VMEM_SHARED((N_PAD, 32), jnp.float32),
        pltpu.VMEM_SHARED((N_PAD, 16), jnp.float32),
    ],
    compiler_params=_sc_params,
)


def _sc_edge2_body(tab, srcs, dsts, vs, sp,
                   src_vm, dst_vm, v_vm, rows0, rows1, msg0, msg1,
                   sem0, sem1, smsg0, smsg1, acc_sh):
    c = lax.axis_index("c")
    s = lax.axis_index("s")
    wid = s * NC + c
    rows_bufs = (rows0, rows1)
    msg_bufs = (msg0, msg1)
    sems = (sem0, sem1)
    smsgs = (smsg0, smsg1)
    msg = msg0

    def init_row(i, _):
        msg[i, pl.ds(0, 16)] = _zero16()
        msg[i, pl.ds(16, 16)] = _zero16()
        msg[i, pl.ds(32, 16)] = _zero16()
        msg[i, pl.ds(48, 16)] = _zero16()
        return 0
    lax.fori_loop(0, MC, init_row, 0)

    for r in range(RPT // MC):
        pltpu.sync_copy(msg, acc_sh.at[pl.ds(s * RPT + r * MC, MC)])
    plsc.subcore_barrier()

    pltpu.sync_copy(srcs.at[wid], src_vm)
    pltpu.sync_copy(dsts.at[wid], dst_vm)
    pltpu.sync_copy(vs.at[wid], v_vm)

    pltpu.async_copy(tab.at[src_vm.at[0]], rows0, sem0)

    def pair(t, _):
        for b in range(2):
            j = t * 2 + b
            rows = rows_bufs[b]
            msg = msg_bufs[b]
            pltpu.make_async_copy(tab.at[src_vm.at[j]], rows, sems[b]).wait()

            @pl.when(j + 1 < NCH)
            def _prefetch():
                pltpu.async_copy(tab.at[src_vm.at[j + 1]],
                                 rows_bufs[1 - b], sems[1 - b])


        return 0
    lax.fori_loop(0, NCH // 2, pair, 0)

    plsc.subcore_barrier()
    pltpu.sync_copy(acc_sh.at[pl.ds(s * RPT, RPT)], sp.at[c, pl.ds(s * RPT, RPT)])


_sc_edge2 = pl.kernel(
    _sc_edge2_body,
    out_type=jax.ShapeDtypeStruct((NC, N_PAD, 64), jnp.float32),
    mesh=_mesh,
    scratch_types=[
        pltpu.VMEM((NCH, MC), jnp.int32),
        pltpu.VMEM((NCH, MC), jnp.int32),
        pltpu.VMEM((NCH, MC), jnp.float32),
        pltpu.VMEM((MC, 32), jnp.float32),
        pltpu.VMEM((MC, 32), jnp.float32),
        pltpu.VMEM((MC, 64), jnp.float32),
        pltpu.VMEM((MC, 64), jnp.float32),
        pltpu.SemaphoreType.DMA,
        pltpu.SemaphoreType.DMA,
        pltpu.SemaphoreType.DMA,
        pltpu.SemaphoreType.DMA,
        pltpu.VMEM_SHARED((N_PAD, 64), jnp.float32),
    ],
    compiler_params=_sc_params,
)


def _tc_pre_body(x_ref, wcat_ref, r1_ref, b1_ref, tab_ref, xr_ref):
    xb = x_ref[...]
    tab_ref[...] = jnp.dot(xb, wcat_ref[...], preferred_element_type=jnp.float32)
    xr_ref[...] = jnp.dot(xb, r1_ref[...], preferred_element_type=jnp.float32) + b1_ref[...]


_tc_pre = pl.pallas_call(
    _tc_pre_body,
    grid=(GRID,),
    in_specs=[
        pl.BlockSpec((BLK, D_IN), lambda i: (i, 0)),
        pl.BlockSpec((D_IN, 64), lambda i: (0, 0)),
        pl.BlockSpec((D_IN, H), lambda i: (0, 0)),
        pl.BlockSpec((1, H), lambda i: (0, 0)),
    ],
    out_specs=[
        pl.BlockSpec((BLK, 64), lambda i: (i, 0)),
        pl.BlockSpec((BLK, H), lambda i: (i, 0)),
    ],
    out_shape=[
        jax.ShapeDtypeStruct((N_PAD, 64), jnp.float32),
        jax.ShapeDtypeStruct((N_PAD, H), jnp.float32),
    ],
)


def _deg(cntp_blk):
    d = jnp.sum(cntp_blk[0] + cntp_blk[1], axis=1, keepdims=True) * (1.0 / 16.0)
    return jnp.maximum(d, 1.0)


def _tc_mid_body(aggp_ref, cntp_ref, xr_ref, r2_ref, b2_ref, h_ref, base_ref):
    aggp = aggp_ref[...]
    agg = (aggp[0] + aggp[1]) / _deg(cntp_ref[...])
    h = jnp.maximum(agg + xr_ref[...], 0.0)
    h_ref[...] = h
    base_ref[...] = jnp.dot(h, r2_ref[...], preferred_element_type=jnp.float32) + b2_ref[...]


_tc_mid = pl.pallas_call(
    _tc_mid_body,
    grid=(GRID,),
    in_specs=[
        pl.BlockSpec((NC, BLK, 32), lambda i: (0, i, 0)),
        pl.BlockSpec((NC, BLK, 16), lambda i: (0, i, 0)),
        pl.BlockSpec((BLK, H), lambda i: (i, 0)),
        pl.BlockSpec((H, D_OUT), lambda i: (0, 0)),
        pl.BlockSpec((1, D_OUT), lambda i: (0, 0)),
    ],
    out_specs=[
        pl.BlockSpec((BLK, H), lambda i: (i, 0)),
        pl.BlockSpec((BLK, D_OUT), lambda i: (i, 0)),
    ],
    out_shape=[
        jax.ShapeDtypeStruct((N_PAD, H), jnp.float32),
        jax.ShapeDtypeStruct((N_PAD, D_OUT), jnp.float32),
    ],
)


def _tc_out_body(sp_ref, cntp_ref, w2_ref, base_ref, out_ref):
    sp = sp_ref[...]
    sblk = (sp[0] + sp[1]) / _deg(cntp_ref[...])
    out_ref[...] = jnp.dot(sblk, w2_ref[...], preferred_element_type=jnp.float32) + base_ref[...]


_tc_out = pl.pallas_call(
    _tc_out_body,
    grid=(GRID,),
    in_specs=[
        pl.BlockSpec((NC, BLK, 64), lambda i: (0, i, 0)),
        pl.BlockSpec((NC, BLK, 16), lambda i: (0, i, 0)),
        pl.BlockSpec((2 * H, D_OUT), lambda i: (0, 0)),
        pl.BlockSpec((BLK, D_OUT), lambda i: (i, 0)),
    ],
    out_specs=pl.BlockSpec((BLK, D_OUT), lambda i: (i, 0)),
    out_shape=jax.ShapeDtypeStruct((N_PAD, D_OUT), jnp.float32),
)


@jax.jit
def _run(x, edge_index, edge_attr, W1, R1, b1, W2, R2, b2):
    src = edge_index[0]
    dst = edge_index[1]
    v = edge_attr[:, 0]
    pad_e = E_PAD - E
    srcp = jnp.concatenate([src, jnp.zeros((pad_e,), jnp.int32)]).reshape(NW, NCH, MC)
    dstp = jnp.concatenate([dst, jnp.full((pad_e,), N, jnp.int32)]).reshape(NW, NCH, MC)
    vp = jnp.concatenate([v, jnp.zeros((pad_e,), jnp.float32)]).reshape(NW, NCH, MC)
    xpad = jnp.pad(x, ((0, N_PAD - N), (0, 0)))

    wcat = jnp.concatenate([W1[0], W1[1] - W1[0]], axis=1)
    w2cat = W2.reshape(2 * H, D_OUT)

    tab1, xr = _tc_pre(xpad, wcat, R1, b1.reshape(1, H))
    aggp, cntp = _sc_edge1(tab1, srcp, dstp, vp)
    h, base = _tc_mid(aggp, cntp, xr, R2, b2.reshape(1, D_OUT))
    sp = _sc_edge2(h, srcp, dstp, vp)
    out = _tc_out(sp, cntp, w2cat, base)
    return out[:N]


def kernel(x, edge_index, edge_attr, W1, R1, b1, W2, R2, b2):
    return _run(x, edge_index, edge_attr, W1, R1, b1, W2, R2, b2)


# speedup vs baseline: 1.7997x; 1.7997x over previous
import functools

import jax
import jax.numpy as jnp
from jax import lax
from jax.experimental import pallas as pl
from jax.experimental.pallas import tpu as pltpu
from jax.experimental.pallas import tpu_sc as plsc

N = 10000
E = 320000
D_IN = 128
H = 32
D_OUT = 128

NC = 2
NS = 16
NW = NC * NS
MC = 128
NCH = 80
EPW = NCH * MC
E_PAD = NW * EPW
N_PAD = 10240
RPT = N_PAD // NS

BLK = 1280
GRID = N_PAD // BLK

_mesh = plsc.VectorSubcoreMesh(core_axis_name="c", subcore_axis_name="s")
_sc_params = pltpu.CompilerParams(use_tc_tiling_on_sc=False)


def _zero16():
    return jnp.zeros((16,), jnp.float32)


def _sc_edge1_body(tab, srcs, dsts, vs, aggp, cntp,
                   src_vm, dst_vm, v_vm, rows0, rows1, msg0, msg1, ones, zer16,
                   sem0, sem1, smsg0, smsg1, sones, acc_sh, cnt_sh, tab_sh):
    c = lax.axis_index("c")
    s = lax.axis_index("s")
    wid = s * NC + c
    rows_bufs = (rows0, rows1)
    msg_bufs = (msg0, msg1)
    sems = (sem0, sem1)
    smsgs = (smsg0, smsg1)
    msg = msg0

    one16 = jnp.ones((16,), jnp.float32)

    def init_row(i, _):
        msg[i, pl.ds(0, 16)] = _zero16()
        msg[i, pl.ds(16, 16)] = _zero16()
        ones[i, pl.ds(0, 16)] = one16
        zer16[i, pl.ds(0, 16)] = _zero16()
        return 0
    lax.fori_loop(0, MC, init_row, 0)

    for r in range(RPT // MC):
        base = s * RPT + r * MC
        pltpu.sync_copy(msg, acc_sh.at[pl.ds(base, MC)])
        pltpu.sync_copy(zer16, cnt_sh.at[pl.ds(base, MC)])
    pltpu.sync_copy(tab.at[pl.ds(s * RPT, RPT)], tab_sh.at[pl.ds(s * RPT, RPT)])
    plsc.subcore_barrier()

    pltpu.sync_copy(srcs.at[wid], src_vm)
    pltpu.sync_copy(dsts.at[wid], dst_vm)
    pltpu.sync_copy(vs.at[wid], v_vm)

    pltpu.async_copy(tab_sh.at[src_vm.at[0]], rows0, sem0)

    def pair(t, _):
        for b in range(2):
            j = t * 2 + b
            rows = rows_bufs[b]
            msg = msg_bufs[b]
            pltpu.make_async_copy(tab_sh.at[src_vm.at[j]], rows, sems[b]).wait()

            @pl.when(j + 1 < NCH)
            def _prefetch():
                pltpu.async_copy(tab_sh.at[src_vm.at[j + 1]],
                                 rows_bufs[1 - b], sems[1 - b])

            @pl.when(j >= 2)
            def _drain_msg():
                pltpu.make_async_copy(msg, acc_sh.at[dst_vm.at[j]],
                                      smsgs[b]).wait()

            def grp(g, _):
                v16 = v_vm[j, pl.ds(g * 16, 16)]
                v16 = jnp.minimum(jnp.maximum(v16, 0.0), 1.0)
                r0 = g * 16
                for e in range(16):
                    ve = v16[e]
                    a0 = rows[r0 + e, pl.ds(0, 16)]
                    a1 = rows[r0 + e, pl.ds(16, 16)]
                    d0 = rows[r0 + e, pl.ds(32, 16)]
                    d1 = rows[r0 + e, pl.ds(48, 16)]
                    msg[r0 + e, pl.ds(0, 16)] = a0 + ve * d0
                    msg[r0 + e, pl.ds(16, 16)] = a1 + ve * d1
                return 0
            lax.fori_loop(0, MC // 16, grp, 0)

            pltpu.async_copy(msg, acc_sh.at[dst_vm.at[j]], smsgs[b], add=True)
            pltpu.async_copy(ones, cnt_sh.at[dst_vm.at[j]], sones, add=True)

            @pl.when(j >= 1)
            def _drain_ones():
                pltpu.make_async_copy(ones, cnt_sh.at[dst_vm.at[j]],
                                      sones).wait()
        return 0
    lax.fori_loop(0, NCH // 2, pair, 0)
    pltpu.make_async_copy(msg0, acc_sh.at[dst_vm.at[0]], smsg0).wait()
    pltpu.make_async_copy(msg1, acc_sh.at[dst_vm.at[0]], smsg1).wait()
    pltpu.make_async_copy(ones, cnt_sh.at[dst_vm.at[0]], sones).wait()

    plsc.subcore_barrier()
    pltpu.sync_copy(acc_sh.at[pl.ds(s * RPT, RPT)], aggp.at[c, pl.ds(s * RPT, RPT)])
    pltpu.sync_copy(cnt_sh.at[pl.ds(s * RPT, RPT)], cntp.at[c, pl.ds(s * RPT, RPT)])


_sc_edge1 = pl.kernel(
    _sc_edge1_body,
    out_type=(jax.ShapeDtypeStruct((NC, N_PAD, 32), jnp.float32),
              jax.ShapeDtypeStruct((NC, N_PAD, 16), jnp.float32)),
    mesh=_mesh,
    scratch_types=[
        pltpu.VMEM((NCH, MC), jnp.int32),
        pltpu.VMEM((NCH, MC), jnp.int32),
        pltpu.VMEM((NCH, MC), jnp.float32),
        pltpu.VMEM((MC, 64), jnp.float32),
        pltpu.VMEM((MC, 64), jnp.float32),
        pltpu.VMEM((MC, 32), jnp.float32),
        pltpu.VMEM((MC, 32), jnp.float32),
        pltpu.VMEM((MC, 16), jnp.float32),
        pltpu.VMEM((MC, 16), jnp.float32),
        pltpu.SemaphoreType.DMA,
        pltpu.SemaphoreType.DMA,
        pltpu.SemaphoreType.DMA,
        pltpu.SemaphoreType.DMA,
        pltpu.SemaphoreType.DMA,
        pltpu.VMEM_SHARED((N_PAD, 32), jnp.float32),
        pltpu.VMEM_SHARED((N_PAD, 16), jnp.float32),
        pltpu.VMEM_SHARED((N_PAD, 64), jnp.float32),
    ],
    compiler_params=_sc_params,
)


def _sc_edge2_body(tab, srcs, dsts, vs, sp,
                   src_vm, dst_vm, v_vm, rows0, rows1, msg0, msg1,
                   sem0, sem1, smsg0, smsg1, acc_sh, tab_sh):
    c = lax.axis_index("c")
    s = lax.axis_index("s")
    wid = s * NC + c
    rows_bufs = (rows0, rows1)
    msg_bufs = (msg0, msg1)
    sems = (sem0, sem1)
    smsgs = (smsg0, smsg1)
    msg = msg0

    def init_row(i, _):
        msg[i, pl.ds(0, 16)] = _zero16()
        msg[i, pl.ds(16, 16)] = _zero16()
        msg[i, pl.ds(32, 16)] = _zero16()
        msg[i, pl.ds(48, 16)] = _zero16()
        return 0
    lax.fori_loop(0, MC, init_row, 0)

    for r in range(RPT // MC):
        pltpu.sync_copy(msg, acc_sh.at[pl.ds(s * RPT + r * MC, MC)])
    pltpu.sync_copy(tab.at[pl.ds(s * RPT, RPT)], tab_sh.at[pl.ds(s * RPT, RPT)])
    plsc.subcore_barrier()

    pltpu.sync_copy(srcs.at[wid], src_vm)
    pltpu.sync_copy(dsts.at[wid], dst_vm)
    pltpu.sync_copy(vs.at[wid], v_vm)

    pltpu.async_copy(tab_sh.at[src_vm.at[0]], rows0, sem0)

    def pair(t, _):
        for b in range(2):
            j = t * 2 + b
            rows = rows_bufs[b]
            msg = msg_bufs[b]
            pltpu.make_async_copy(tab_sh.at[src_vm.at[j]], rows, sems[b]).wait()

            @pl.when(j + 1 < NCH)
            def _prefetch():
                pltpu.async_copy(tab_sh.at[src_vm.at[j + 1]],
                                 rows_bufs[1 - b], sems[1 - b])

            @pl.when(j >= 2)
            def _drain_msg():
                pltpu.make_async_copy(msg, acc_sh.at[dst_vm.at[j]],
                                      smsgs[b]).wait()

            def grp(g, _):
                v16 = v_vm[j, pl.ds(g * 16, 16)]
                v16 = jnp.minimum(jnp.maximum(v16, 0.0), 1.0)
                r0 = g * 16
                for e in range(16):
                    ve = v16[e]
                    a0 = rows[r0 + e, pl.ds(0, 16)]
                    a1 = rows[r0 + e, pl.ds(16, 16)]
                    t0 = ve * a0
                    t1 = ve * a1
                    msg[r0 + e, pl.ds(0, 16)] = a0 - t0
                    msg[r0 + e, pl.ds(16, 16)] = a1 - t1
                    msg[r0 + e, pl.ds(32, 16)] = t0
                    msg[r0 + e, pl.ds(48, 16)] = t1
                return 0
            lax.fori_loop(0, MC // 16, grp, 0)

            pltpu.async_copy(msg, acc_sh.at[dst_vm.at[j]], smsgs[b], add=True)
        return 0
    lax.fori_loop(0, NCH // 2, pair, 0)
    pltpu.make_async_copy(msg0, acc_sh.at[dst_vm.at[0]], smsg0).wait()
    pltpu.make_async_copy(msg1, acc_sh.at[dst_vm.at[0]], smsg1).wait()

    plsc.subcore_barrier()
    pltpu.sync_copy(acc_sh.at[pl.ds(s * RPT, RPT)], sp.at[c, pl.ds(s * RPT, RPT)])


_sc_edge2 = pl.kernel(
    _sc_edge2_body,
    out_type=jax.ShapeDtypeStruct((NC, N_PAD, 64), jnp.float32),
    mesh=_mesh,
    scratch_types=[
        pltpu.VMEM((NCH, MC), jnp.int32),
        pltpu.VMEM((NCH, MC), jnp.int32),
        pltpu.VMEM((NCH, MC), jnp.float32),
        pltpu.VMEM((MC, 32), jnp.float32),
        pltpu.VMEM((MC, 32), jnp.float32),
        pltpu.VMEM((MC, 64), jnp.float32),
        pltpu.VMEM((MC, 64), jnp.float32),
        pltpu.SemaphoreType.DMA,
        pltpu.SemaphoreType.DMA,
        pltpu.SemaphoreType.DMA,
        pltpu.SemaphoreType.DMA,
        pltpu.VMEM_SHARED((N_PAD, 64), jnp.float32),
        pltpu.VMEM_SHARED((N_PAD, 32), jnp.float32),
    ],
    compiler_params=_sc_params,
)


def _tc_pre_body(x_ref, wcat_ref, r1_ref, b1_ref, tab_ref, xr_ref):
    xb = x_ref[...]
    tab_ref[...] = jnp.dot(xb, wcat_ref[...], preferred_element_type=jnp.float32)
    xr_ref[...] = jnp.dot(xb, r1_ref[...], preferred_element_type=jnp.float32) + b1_ref[...]


_tc_pre = pl.pallas_call(
    _tc_pre_body,
    grid=(GRID,),
    in_specs=[
        pl.BlockSpec((BLK, D_IN), lambda i: (i, 0)),
        pl.BlockSpec((D_IN, 64), lambda i: (0, 0)),
        pl.BlockSpec((D_IN, H), lambda i: (0, 0)),
        pl.BlockSpec((1, H), lambda i: (0, 0)),
    ],
    out_specs=[
        pl.BlockSpec((BLK, 64), lambda i: (i, 0)),
        pl.BlockSpec((BLK, H), lambda i: (i, 0)),
    ],
    out_shape=[
        jax.ShapeDtypeStruct((N_PAD, 64), jnp.float32),
        jax.ShapeDtypeStruct((N_PAD, H), jnp.float32),
    ],
)


def _deg(cntp_blk):
    d = jnp.sum(cntp_blk[0] + cntp_blk[1], axis=1, keepdims=True) * (1.0 / 16.0)
    return jnp.maximum(d, 1.0)


def _tc_mid_body(aggp_ref, cntp_ref, xr_ref, r2_ref, b2_ref, h_ref, base_ref):
    aggp = aggp_ref[...]
    agg = (aggp[0] + aggp[1]) / _deg(cntp_ref[...])
    h = jnp.maximum(agg + xr_ref[...], 0.0)
    h_ref[...] = h
    base_ref[...] = jnp.dot(h, r2_ref[...], preferred_element_type=jnp.float32) + b2_ref[...]


_tc_mid = pl.pallas_call(
    _tc_mid_body,
    grid=(GRID,),
    in_specs=[
        pl.BlockSpec((NC, BLK, 32), lambda i: (0, i, 0)),
        pl.BlockSpec((NC, BLK, 16), lambda i: (0, i, 0)),
        pl.BlockSpec((BLK, H), lambda i: (i, 0)),
        pl.BlockSpec((H, D_OUT), lambda i: (0, 0)),
        pl.BlockSpec((1, D_OUT), lambda i: (0, 0)),
    ],
    out_specs=[
        pl.BlockSpec((BLK, H), lambda i: (i, 0)),
        pl.BlockSpec((BLK, D_OUT), lambda i: (i, 0)),
    ],
    out_shape=[
        jax.ShapeDtypeStruct((N_PAD, H), jnp.float32),
        jax.ShapeDtypeStruct((N_PAD, D_OUT), jnp.float32),
    ],
)


def _tc_out_body(sp_ref, cntp_ref, w2_ref, base_ref, out_ref):
    sp = sp_ref[...]
    sblk = (sp[0] + sp[1]) / _deg(cntp_ref[...])
    out_ref[...] = jnp.dot(sblk, w2_ref[...], preferred_element_type=jnp.float32) + base_ref[...]


_tc_out = pl.pallas_call(
    _tc_out_body,
    grid=(GRID,),
    in_specs=[
        pl.BlockSpec((NC, BLK, 64), lambda i: (0, i, 0)),
        pl.BlockSpec((NC, BLK, 16), lambda i: (0, i, 0)),
        pl.BlockSpec((2 * H, D_OUT), lambda i: (0, 0)),
        pl.BlockSpec((BLK, D_OUT), lambda i: (i, 0)),
    ],
    out_specs=pl.BlockSpec((BLK, D_OUT), lambda i: (i, 0)),
    out_shape=jax.ShapeDtypeStruct((N_PAD, D_OUT), jnp.float32),
)


@jax.jit
def _run(x, edge_index, edge_attr, W1, R1, b1, W2, R2, b2):
    src = edge_index[0]
    dst = edge_index[1]
    v = edge_attr[:, 0]
    pad_e = E_PAD - E
    srcp = jnp.concatenate([src, jnp.zeros((pad_e,), jnp.int32)]).reshape(NW, NCH, MC)
    dstp = jnp.concatenate([dst, jnp.full((pad_e,), N, jnp.int32)]).reshape(NW, NCH, MC)
    vp = jnp.concatenate([v, jnp.zeros((pad_e,), jnp.float32)]).reshape(NW, NCH, MC)
    xpad = jnp.pad(x, ((0, N_PAD - N), (0, 0)))

    wcat = jnp.concatenate([W1[0], W1[1] - W1[0]], axis=1)
    w2cat = W2.reshape(2 * H, D_OUT)

    tab1, xr = _tc_pre(xpad, wcat, R1, b1.reshape(1, H))
    aggp, cntp = _sc_edge1(tab1, srcp, dstp, vp)
    h, base = _tc_mid(aggp, cntp, xr, R2, b2.reshape(1, D_OUT))
    sp = _sc_edge2(h, srcp, dstp, vp)
    out = _tc_out(sp, cntp, w2cat, base)
    return out[:N]


def kernel(x, edge_index, edge_attr, W1, R1, b1, W2, R2, b2):
    return _run(x, edge_index, edge_attr, W1, R1, b1, W2, R2, b2)
